# Initial kernel scaffold; baseline (speedup 1.0000x reference)
import jax
import jax.numpy as jnp
from jax.experimental import pallas as pl


def kernel(x, routing_assignments, W1, b1, W2, b2):
    raise NotImplementedError("write your pallas kernel here")


# speedup vs baseline: 1.6738x; 1.6738x over previous
import jax
import jax.numpy as jnp
from jax.experimental import pallas as pl
from jax.experimental.pallas import tpu as pltpu

_TM = 512
_TH = 2048


def _ffn_body(te_ref, tf_ref, x_ref, w1_ref, b1_ref, w2_ref, b2_ref, o_ref):
    t = pl.program_id(0)
    hc = pl.program_id(1)
    nhc = pl.num_programs(1)

    @pl.when(hc == 0)
    def _init():
        o_ref[...] = jnp.zeros_like(o_ref)

    @pl.when(tf_ref[t] > 0)
    def _compute():
        xb = x_ref[...].astype(jnp.bfloat16)
        w1 = w1_ref[0].astype(jnp.bfloat16)
        h = jnp.dot(xb, w1, preferred_element_type=jnp.float32)
        h = jnp.maximum(h + b1_ref[0], 0.0).astype(jnp.bfloat16)
        w2 = w2_ref[0].astype(jnp.bfloat16)
        o_ref[...] += jnp.dot(h, w2, preferred_element_type=jnp.float32)

    @pl.when(jnp.logical_and(tf_ref[t] > 0, hc == nhc - 1))
    def _bias():
        o_ref[...] += b2_ref[0]


def _grouped_ffn(x_sorted, W1, b1, W2, b2, tile_expert, tile_flag):
    NP, D = x_sorted.shape
    H = W1.shape[2]
    NT = NP // _TM
    HC = H // _TH
    return pl.pallas_call(
        _ffn_body,
        grid_spec=pltpu.PrefetchScalarGridSpec(
            num_scalar_prefetch=2,
            grid=(NT, HC),
            in_specs=[
                pl.BlockSpec((_TM, D), lambda t, hc, te, tf: (t, 0)),
                pl.BlockSpec((1, D, _TH), lambda t, hc, te, tf: (te[t], 0, hc)),
                pl.BlockSpec((1, 1, _TH), lambda t, hc, te, tf: (te[t], 0, hc)),
                pl.BlockSpec((1, _TH, D), lambda t, hc, te, tf: (te[t], hc, 0)),
                pl.BlockSpec((1, 1, D), lambda t, hc, te, tf: (te[t], 0, 0)),
            ],
            out_specs=pl.BlockSpec((_TM, D), lambda t, hc, te, tf: (t, 0)),
        ),
        out_shape=jax.ShapeDtypeStruct((NP, D), jnp.float32),
        compiler_params=pltpu.CompilerParams(
            dimension_semantics=("arbitrary", "arbitrary"),
        ),
    )(tile_expert, tile_flag, x_sorted, W1,
      b1.reshape(b1.shape[0], 1, H), W2, b2.reshape(b2.shape[0], 1, D))


def kernel(x, routing_assignments, W1, b1, W2, b2):
    B, S, D = x.shape
    E = W1.shape[0]
    N = B * S
    NT = N // _TM + E - 1

    x_flat = x.reshape(N, D)
    assign = routing_assignments.astype(jnp.int32)

    oh = assign[:, None] == jnp.arange(E, dtype=jnp.int32)[None, :]
    ohi = oh.astype(jnp.int32)
    counts = jnp.sum(ohi, axis=0)
    rank = jnp.sum(jnp.where(oh, jnp.cumsum(ohi, axis=0) - 1, 0), axis=1)
    nt_e = (counts + _TM - 1) // _TM
    tile_off = jnp.concatenate(
        [jnp.zeros((1,), jnp.int32), jnp.cumsum(nt_e, dtype=jnp.int32)])
    pos = tile_off[assign] * _TM + rank
    used = tile_off[E]
    tidx = jnp.arange(NT, dtype=jnp.int32)
    te_raw = jnp.sum(
        (tidx[:, None] >= tile_off[None, 1:]).astype(jnp.int32), axis=1)
    last_e = jnp.max(jnp.where(tidx < used, te_raw, -1))
    tile_expert = jnp.where(tidx < used, te_raw, last_e).astype(jnp.int32)
    tile_flag = (tidx < used).astype(jnp.int32)
    gather_src = jnp.zeros((NT * _TM,), jnp.int32).at[pos].set(
        jnp.arange(N, dtype=jnp.int32))

    x_sorted = jnp.take(x_flat, gather_src, axis=0)
    out_sorted = _grouped_ffn(x_sorted, W1, b1, W2, b2, tile_expert, tile_flag)
    out = jnp.take(out_sorted, pos, axis=0)
    return out.reshape(B, S, D)


# speedup vs baseline: 3.2994x; 1.9713x over previous
import functools

import jax
import jax.numpy as jnp
from jax import lax
from jax.experimental import pallas as pl
from jax.experimental.pallas import tpu as pltpu
from jax.experimental.pallas import tpu_sc as plsc

_TM = 512
_TH = 2048
_NC, _NS = 2, 16
_NW = _NC * _NS
_CH = 32


def _sc_scatter_rows(x_flat, pos3, NP):
    N, D = x_flat.shape
    bn = N // _NW
    nchunk = bn // _CH
    mesh = plsc.VectorSubcoreMesh(core_axis_name="c", subcore_axis_name="s")

    @functools.partial(
        pl.kernel,
        out_type=jax.ShapeDtypeStruct((NP, D), jnp.float32),
        mesh=mesh,
        scratch_types=[
            pltpu.VMEM((nchunk, _CH), jnp.int32),
            pltpu.VMEM((_CH, D), jnp.float32),
            pltpu.VMEM((_CH, D), jnp.float32),
            pltpu.SemaphoreType.DMA,
            pltpu.SemaphoreType.DMA,
        ],
    )
    def k(x_hbm, pos_hbm, out_hbm, idx_v, buf0, buf1, sem_in, sem_out):
        wid = lax.axis_index("s") * _NC + lax.axis_index("c")
        base = wid * bn
        pltpu.sync_copy(pos_hbm.at[wid], idx_v)
        bufs = (buf0, buf1)
        h = pltpu.async_copy(x_hbm.at[pl.ds(base, _CH)], bufs[0], sem_in)
        for j in range(nchunk):
            b = bufs[j % 2]
            h.wait()
            if j + 1 < nchunk:
                h = pltpu.async_copy(
                    x_hbm.at[pl.ds(base + (j + 1) * _CH, _CH)],
                    bufs[(j + 1) % 2], sem_in)
            pltpu.async_copy(b, out_hbm.at[idx_v.at[j]], sem_out).wait()

    return k(x_flat, pos3)


def _sc_gather_rows(table, pos3, N):
    D = table.shape[1]
    bn = N // _NW
    nchunk = bn // _CH
    mesh = plsc.VectorSubcoreMesh(core_axis_name="c", subcore_axis_name="s")

    @functools.partial(
        pl.kernel,
        out_type=jax.ShapeDtypeStruct((N, D), jnp.float32),
        mesh=mesh,
        scratch_types=[
            pltpu.VMEM((nchunk, _CH), jnp.int32),
            pltpu.VMEM((_CH, D), jnp.float32),
            pltpu.VMEM((_CH, D), jnp.float32),
            pltpu.SemaphoreType.DMA,
            pltpu.SemaphoreType.DMA,
        ],
    )
    def k(tab_hbm, pos_hbm, out_hbm, idx_v, buf0, buf1, sem_in, sem_out):
        wid = lax.axis_index("s") * _NC + lax.axis_index("c")
        base = wid * bn
        pltpu.sync_copy(pos_hbm.at[wid], idx_v)
        bufs = (buf0, buf1)
        h = pltpu.async_copy(tab_hbm.at[idx_v.at[0]], bufs[0], sem_in)
        for j in range(nchunk):
            b = bufs[j % 2]
            h.wait()
            if j + 1 < nchunk:
                h = pltpu.async_copy(
                    tab_hbm.at[idx_v.at[j + 1]], bufs[(j + 1) % 2], sem_in)
            pltpu.async_copy(
                b, out_hbm.at[pl.ds(base + j * _CH, _CH)], sem_out).wait()

    return k(table, pos3)


def _ffn_body(te_ref, tf_ref, x_ref, w1_ref, b1_ref, w2_ref, b2_ref, o_ref):
    t = pl.program_id(0)
    hc = pl.program_id(1)
    nhc = pl.num_programs(1)

    @pl.when(hc == 0)
    def _init():
        o_ref[...] = jnp.zeros_like(o_ref)

    @pl.when(tf_ref[t] > 0)
    def _compute():
        xb = x_ref[...].astype(jnp.bfloat16)
        w1 = w1_ref[0].astype(jnp.bfloat16)
        h = jnp.dot(xb, w1, preferred_element_type=jnp.float32)
        h = jnp.maximum(h + b1_ref[0], 0.0).astype(jnp.bfloat16)
        w2 = w2_ref[0].astype(jnp.bfloat16)
        o_ref[...] += jnp.dot(h, w2, preferred_element_type=jnp.float32)

    @pl.when(jnp.logical_and(tf_ref[t] > 0, hc == nhc - 1))
    def _bias():
        o_ref[...] += b2_ref[0]


def _grouped_ffn(x_sorted, W1, b1, W2, b2, tile_expert, tile_flag):
    NP, D = x_sorted.shape
    H = W1.shape[2]
    NT = NP // _TM
    HC = H // _TH
    return pl.pallas_call(
        _ffn_body,
        grid_spec=pltpu.PrefetchScalarGridSpec(
            num_scalar_prefetch=2,
            grid=(NT, HC),
            in_specs=[
                pl.BlockSpec((_TM, D), lambda t, hc, te, tf: (t, 0)),
                pl.BlockSpec((1, D, _TH), lambda t, hc, te, tf: (te[t], 0, hc)),
                pl.BlockSpec((1, 1, _TH), lambda t, hc, te, tf: (te[t], 0, hc)),
                pl.BlockSpec((1, _TH, D), lambda t, hc, te, tf: (te[t], hc, 0)),
                pl.BlockSpec((1, 1, D), lambda t, hc, te, tf: (te[t], 0, 0)),
            ],
            out_specs=pl.BlockSpec((_TM, D), lambda t, hc, te, tf: (t, 0)),
        ),
        out_shape=jax.ShapeDtypeStruct((NP, D), jnp.float32),
        compiler_params=pltpu.CompilerParams(
            dimension_semantics=("arbitrary", "arbitrary"),
        ),
    )(tile_expert, tile_flag, x_sorted, W1,
      b1.reshape(b1.shape[0], 1, H), W2, b2.reshape(b2.shape[0], 1, D))


def kernel(x, routing_assignments, W1, b1, W2, b2):
    B, S, D = x.shape
    E = W1.shape[0]
    N = B * S
    NT = N // _TM + E - 1
    NP = NT * _TM

    x_flat = x.reshape(N, D)
    assign = routing_assignments.astype(jnp.int32)

    oh = assign[:, None] == jnp.arange(E, dtype=jnp.int32)[None, :]
    ohi = oh.astype(jnp.int32)
    counts = jnp.sum(ohi, axis=0)
    rank = jnp.sum(jnp.where(oh, jnp.cumsum(ohi, axis=0) - 1, 0), axis=1)
    nt_e = (counts + _TM - 1) // _TM
    tile_off = jnp.concatenate(
        [jnp.zeros((1,), jnp.int32), jnp.cumsum(nt_e, dtype=jnp.int32)])
    pos = tile_off[assign] * _TM + rank
    used = tile_off[E]
    tidx = jnp.arange(NT, dtype=jnp.int32)
    te_raw = jnp.sum(
        (tidx[:, None] >= tile_off[None, 1:]).astype(jnp.int32), axis=1)
    last_e = jnp.max(jnp.where(tidx < used, te_raw, -1))
    tile_expert = jnp.where(tidx < used, te_raw, last_e).astype(jnp.int32)
    tile_flag = (tidx < used).astype(jnp.int32)

    pos3 = pos.reshape(_NW, (N // _NW) // _CH, _CH)
    x_sorted = _sc_scatter_rows(x_flat, pos3, NP)
    out_sorted = _grouped_ffn(x_sorted, W1, b1, W2, b2, tile_expert, tile_flag)
    out = _sc_gather_rows(out_sorted, pos3, N)
    return out.reshape(B, S, D)
